# baseline (device time: 12027 ns/iter reference)
import jax
import jax.numpy as jnp
from jax import lax
from jax.experimental import pallas as pl
from jax.experimental.pallas import tpu as pltpu

N_CHUNK = 8


def kernel(x):
    m, n = x.shape
    qm = m // 4
    cm = qm // N_CHUNK

    def body(
        x_ref,
        out_ref,
        a_bf,
        b_bf,
        recv_a,
        recv_b,
        send_sems,
        recv_sems,
        yz_sem,
    ):
        my_x = lax.axis_index("x")
        my_y = lax.axis_index("y")
        my_z = lax.axis_index("z")
        x_partner = (1 - my_x, my_y, my_z)
        y_partner = (my_x, 1 - my_y, my_z)
        z_partner = (my_x, my_y, 1 - my_z)

        q_own = 2 * my_y + my_z
        q_diag = 3 - q_own

        barrier_sem = pltpu.get_barrier_semaphore()
        pl.semaphore_signal(
            barrier_sem, inc=1,
            device_id=x_partner, device_id_type=pl.DeviceIdType.MESH,
        )
        for nbr in (y_partner, z_partner):
            pl.semaphore_signal(
                yz_sem, inc=1,
                device_id=nbr, device_id_type=pl.DeviceIdType.MESH,
            )
        pl.semaphore_wait(barrier_sem, 1)

        rdma_xa = []
        for c in range(N_CHUNK):
            crows = pl.ds(c * cm, cm)
            a_bf[crows, :] = x_ref[
                pl.ds(q_own * qm + c * cm, cm), :
            ].astype(jnp.bfloat16)
            r = pltpu.make_async_remote_copy(
                src_ref=a_bf.at[crows, :],
                dst_ref=recv_a.at[crows, :],
                send_sem=send_sems.at[c],
                recv_sem=recv_sems.at[c],
                device_id=x_partner,
                device_id_type=pl.DeviceIdType.MESH,
            )
            r.start()
            rdma_xa.append(r)
        b_bf[...] = x_ref[pl.ds(q_diag * qm, qm), :].astype(jnp.bfloat16)
        rdma_xb = pltpu.make_async_remote_copy(
            src_ref=b_bf,
            dst_ref=recv_b,
            send_sem=send_sems.at[N_CHUNK],
            recv_sem=recv_sems.at[N_CHUNK],
            device_id=x_partner,
            device_id_type=pl.DeviceIdType.MESH,
        )
        rdma_xb.start()

        pl.semaphore_wait(yz_sem, 2)

        rdma_yz = []
        for c in range(N_CHUNK):
            rows = pl.ds(q_own * qm + c * cm, cm)
            crows = pl.ds(c * cm, cm)
            rdma_xa[c].wait_recv()
            out_ref[rows, :] = (
                x_ref[rows, :] + recv_a[crows, :].astype(jnp.float32)
            )
            ry = pltpu.make_async_remote_copy(
                src_ref=out_ref.at[rows, :],
                dst_ref=out_ref.at[rows, :],
                send_sem=send_sems.at[N_CHUNK + 1 + 2 * c],
                recv_sem=recv_sems.at[N_CHUNK + 1 + 2 * c],
                device_id=y_partner,
                device_id_type=pl.DeviceIdType.MESH,
            )
            rz = pltpu.make_async_remote_copy(
                src_ref=out_ref.at[rows, :],
                dst_ref=out_ref.at[rows, :],
                send_sem=send_sems.at[N_CHUNK + 2 + 2 * c],
                recv_sem=recv_sems.at[N_CHUNK + 2 + 2 * c],
                device_id=z_partner,
                device_id_type=pl.DeviceIdType.MESH,
            )
            ry.start()
            rz.start()
            rdma_yz.append((ry, rz))

        rdma_xb.wait_recv()
        out_ref[pl.ds(q_diag * qm, qm), :] = (
            x_ref[pl.ds(q_diag * qm, qm), :]
            + recv_b[...].astype(jnp.float32)
        )

        for ry, rz in rdma_yz:
            ry.wait_recv()
            rz.wait_recv()

        for r in rdma_xa:
            r.wait_send()
        rdma_xb.wait_send()
        for ry, rz in rdma_yz:
            ry.wait_send()
            rz.wait_send()

    n_sems = 1 + N_CHUNK + 2 * N_CHUNK
    return pl.pallas_call(
        body,
        out_shape=jax.ShapeDtypeStruct((m, n), x.dtype),
        in_specs=[pl.BlockSpec(memory_space=pltpu.VMEM)],
        out_specs=pl.BlockSpec(memory_space=pltpu.VMEM),
        scratch_shapes=[
            pltpu.VMEM((qm, n), jnp.bfloat16),
            pltpu.VMEM((qm, n), jnp.bfloat16),
            pltpu.VMEM((qm, n), jnp.bfloat16),
            pltpu.VMEM((qm, n), jnp.bfloat16),
            pltpu.SemaphoreType.DMA((n_sems,)),
            pltpu.SemaphoreType.DMA((n_sems,)),
            pltpu.SemaphoreType.REGULAR,
        ],
        compiler_params=pltpu.CompilerParams(collective_id=0),
    )(x)


# device time: 11987 ns/iter; 1.0033x vs baseline; 1.0033x over previous
import jax
import jax.numpy as jnp
from jax import lax
from jax.experimental import pallas as pl
from jax.experimental.pallas import tpu as pltpu

N_CHUNK = 4


def kernel(x):
    m, n = x.shape
    qm = m // 4
    cm = qm // N_CHUNK

    def body(
        x_ref,
        out_ref,
        a_bf,
        b_bf,
        recv_a,
        recv_b,
        send_sems,
        recv_sems,
        yz_sem,
    ):
        my_x = lax.axis_index("x")
        my_y = lax.axis_index("y")
        my_z = lax.axis_index("z")
        x_partner = (1 - my_x, my_y, my_z)
        y_partner = (my_x, 1 - my_y, my_z)
        z_partner = (my_x, my_y, 1 - my_z)

        q_own = 2 * my_y + my_z
        q_diag = 3 - q_own

        barrier_sem = pltpu.get_barrier_semaphore()
        pl.semaphore_signal(
            barrier_sem, inc=1,
            device_id=x_partner, device_id_type=pl.DeviceIdType.MESH,
        )
        for nbr in (y_partner, z_partner):
            pl.semaphore_signal(
                yz_sem, inc=1,
                device_id=nbr, device_id_type=pl.DeviceIdType.MESH,
            )
        a_bf[...] = x_ref[pl.ds(q_own * qm, qm), :].astype(jnp.bfloat16)
        b_bf[...] = x_ref[pl.ds(q_diag * qm, qm), :].astype(jnp.bfloat16)

        pl.semaphore_wait(barrier_sem, 1)

        rdma_xa = []
        for c in range(N_CHUNK):
            crows = pl.ds(c * cm, cm)
            r = pltpu.make_async_remote_copy(
                src_ref=a_bf.at[crows, :],
                dst_ref=recv_a.at[crows, :],
                send_sem=send_sems.at[c],
                recv_sem=recv_sems.at[c],
                device_id=x_partner,
                device_id_type=pl.DeviceIdType.MESH,
            )
            r.start()
            rdma_xa.append(r)
        rdma_xb = pltpu.make_async_remote_copy(
            src_ref=b_bf,
            dst_ref=recv_b,
            send_sem=send_sems.at[N_CHUNK],
            recv_sem=recv_sems.at[N_CHUNK],
            device_id=x_partner,
            device_id_type=pl.DeviceIdType.MESH,
        )
        rdma_xb.start()

        pl.semaphore_wait(yz_sem, 2)

        rdma_yz = []
        for c in range(N_CHUNK):
            rows = pl.ds(q_own * qm + c * cm, cm)
            crows = pl.ds(c * cm, cm)
            rdma_xa[c].wait_recv()
            out_ref[rows, :] = (
                x_ref[rows, :] + recv_a[crows, :].astype(jnp.float32)
            )
            ry = pltpu.make_async_remote_copy(
                src_ref=out_ref.at[rows, :],
                dst_ref=out_ref.at[rows, :],
                send_sem=send_sems.at[N_CHUNK + 1 + 2 * c],
                recv_sem=recv_sems.at[N_CHUNK + 1 + 2 * c],
                device_id=y_partner,
                device_id_type=pl.DeviceIdType.MESH,
            )
            rz = pltpu.make_async_remote_copy(
                src_ref=out_ref.at[rows, :],
                dst_ref=out_ref.at[rows, :],
                send_sem=send_sems.at[N_CHUNK + 2 + 2 * c],
                recv_sem=recv_sems.at[N_CHUNK + 2 + 2 * c],
                device_id=z_partner,
                device_id_type=pl.DeviceIdType.MESH,
            )
            ry.start()
            rz.start()
            rdma_yz.append((ry, rz))

        rdma_xb.wait_recv()
        out_ref[pl.ds(q_diag * qm, qm), :] = (
            x_ref[pl.ds(q_diag * qm, qm), :]
            + recv_b[...].astype(jnp.float32)
        )

        for ry, rz in rdma_yz:
            ry.wait_recv()
            rz.wait_recv()

        for r in rdma_xa:
            r.wait_send()
        rdma_xb.wait_send()
        for ry, rz in rdma_yz:
            ry.wait_send()
            rz.wait_send()

    n_sems = 1 + N_CHUNK + 2 * N_CHUNK
    return pl.pallas_call(
        body,
        out_shape=jax.ShapeDtypeStruct((m, n), x.dtype),
        in_specs=[pl.BlockSpec(memory_space=pltpu.VMEM)],
        out_specs=pl.BlockSpec(memory_space=pltpu.VMEM),
        scratch_shapes=[
            pltpu.VMEM((qm, n), jnp.bfloat16),
            pltpu.VMEM((qm, n), jnp.bfloat16),
            pltpu.VMEM((qm, n), jnp.bfloat16),
            pltpu.VMEM((qm, n), jnp.bfloat16),
            pltpu.SemaphoreType.DMA((n_sems,)),
            pltpu.SemaphoreType.DMA((n_sems,)),
            pltpu.SemaphoreType.REGULAR,
        ],
        compiler_params=pltpu.CompilerParams(collective_id=0),
    )(x)
